# baseline (speedup 1.0000x reference)
import functools

import jax
import jax.numpy as jnp
from jax import lax
from jax.experimental import pallas as pl
from jax.experimental import layout as jlayout
from jax.experimental.pallas import tpu as pltpu
from jax.experimental.pallas import tpu_sc as plsc

VOCAB_SIZE = 1000000
EMBED_DIM = 64
BATCH = 16384
HIST_LEN = 50

_INFO = plsc.get_sparse_core_info()
_NC, _NS = _INFO.num_cores, _INFO.num_subcores
_NW = _NC * _NS
_RPW = BATCH // _NW
_BT = BATCH // 128
_BTPW = _BT // _NW
_CT = 256
_NCHUNK = HIST_LEN * 2
_RING = 2


def _make_sc_gather():
  mesh = plsc.VectorSubcoreMesh(core_axis_name="c", subcore_axis_name="s")

  @functools.partial(
      pl.kernel,
      mesh=mesh,
      compiler_params=pltpu.CompilerParams(use_tc_tiling_on_sc=False,
                                           needs_layout_passes=False),
      out_type=jax.ShapeDtypeStruct((HIST_LEN, 8, 128, 8, 128), jnp.float32),
      scratch_types=[
          pltpu.VMEM((HIST_LEN, _RPW), jnp.int32),
          pltpu.VMEM((_RING, _CT, EMBED_DIM), jnp.float32),
          pltpu.VMEM((_RING, 8, 2, 8, 128), jnp.float32),
          [pltpu.SemaphoreType.DMA] * _RING,
          [pltpu.SemaphoreType.DMA] * _RING,
      ],
  )
  def k(table_hbm, idxt_hbm, out_hbm, idx_v, rows_v, t_v, gsem, ssem):
    wid = lax.axis_index("s") * _NC + lax.axis_index("c")
    b0 = wid * _RPW
    bt0 = wid * _BTPW
    pltpu.sync_copy(idxt_hbm.at[:, pl.ds(b0, _RPW)], idx_v)

    iota = lax.iota(jnp.int32, 16)
    dtv = [g * 2 + iota // 8 for g in range(4)]
    dsv = iota % 8

    def g_copy(c, p):
      h, half = c // 2, c % 2
      return pltpu.make_async_copy(
          table_hbm.at[idx_v.at[h, pl.ds(half * _CT, _CT)]], rows_v.at[p],
          gsem[p])

    def s_copies(c, p):
      h, half = c // 2, c % 2
      off = bt0 + half * 2
      return [
          pltpu.make_async_copy(
              t_v.at[p, dt], out_hbm.at[h, dt, pl.ds(off, 2)], ssem[p])
          for dt in range(8)
      ]

    def transpose(gp, p):
      def tbody(j, _):
        for tt in range(4):
          t = j * 4 + tt
          btv = lax.broadcast(t // 128, (16,))
          bsv = lax.broadcast(t % 128, (16,))
          for g in range(4):
            v = rows_v[gp, t, pl.ds(g * 16, 16)]
            plsc.store_scatter(t_v.at[p], [dtv[g], btv, dsv, bsv], v)
        return 0
      lax.fori_loop(0, _CT // 4, tbody, 0)

    def step(c, q, launch_gather, wait_scatter):
      g_copy(c, q).wait()
      if wait_scatter:
        for d in s_copies(0, q):
          d.wait()
      transpose(q, q)
      for d in s_copies(c, q):
        d.start()
      if launch_gather:
        g_copy(c + _RING, q).start()

    for q in range(_RING):
      g_copy(q, q).start()

    for q in range(_RING):
      step(q, q, True, False)

    def body(s, _):
      for q in range(_RING):
        step(s * _RING + q, q, True, True)
      return 0

    lax.fori_loop(1, _NCHUNK // _RING - 1, body, 0)

    for q in range(_RING):
      step(_NCHUNK - _RING + q, q, False, True)

    for q in range(_RING):
      for d in s_copies(0, q):
        d.wait()

  return k


_sc_gather = _make_sc_gather()


def kernel(tokens, embedding_weight):
  idxt = tokens.T.astype(jnp.int32) * 2
  table = jlayout.with_layout_constraint(
      embedding_weight, jlayout.Layout((0, 1), tiling=((8, 128),)))
  out5 = _sc_gather(table, idxt)
  return out5.transpose(2, 4, 0, 1, 3).reshape(BATCH, HIST_LEN, EMBED_DIM)
